# baseline (speedup 1.0000x reference)
import functools

import jax
import jax.numpy as jnp
from jax.experimental import pallas as pl
from jax.experimental.pallas import tpu as pltpu
from jax.experimental.pallas import tpu_sc as plsc

VOCAB = 1000000
D = 64
TAB_WB = 32768
VOCAB_PAD = 31 * 32768
GATHER_W = 512


def _tab_rows(tabT):
    def body(in_ref, out_ref):
        out_ref[:, 0:D] = in_ref[...].T

    return pl.pallas_call(
        body,
        grid=(VOCAB_PAD // TAB_WB,),
        in_specs=[pl.BlockSpec((D, TAB_WB), lambda i: (0, i))],
        out_specs=pl.BlockSpec((TAB_WB, 128), lambda i: (i, 0)),
        out_shape=jax.ShapeDtypeStruct((VOCAB_PAD, 128), jnp.float32),
        compiler_params=pltpu.CompilerParams(
            dimension_semantics=("parallel",)
        ),
    )(tabT)


def _sc_gather(tab128, idx_flat, n):
    mesh = plsc.VectorSubcoreMesh(core_axis_name="c", subcore_axis_name="s")

    @functools.partial(
        pl.kernel,
        out_type=jax.ShapeDtypeStruct((n, D), jnp.float32),
        mesh=mesh,
        compiler_params=pltpu.CompilerParams(use_tc_tiling_on_sc=False),
    )
    def gather_kernel(x_hbm, i_hbm, o_hbm):
        def body(i_vmem, o_vmem):
            pltpu.sync_copy(x_hbm.at[i_vmem.at[0]], o_vmem)

        pltpu.emit_pipeline(
            body,
            grid=(n // GATHER_W,),
            in_specs=[pl.BlockSpec((1, GATHER_W), lambda i: (0, i))],
            out_specs=[pl.BlockSpec((GATHER_W, D), lambda i: (i, 0))],
            core_axis_name=("c", "s"),
            dimension_semantics=(pltpu.PARALLEL,),
        )(i_hbm, o_hbm)

    return gather_kernel(tab128, idx_flat)


SB = 100


def _out_format(g4, S, NB):
    def body(in_ref, out_ref):
        for j in range(SB):
            a = in_ref[j, 0, :, 0:D]
            b = in_ref[j, 0, :, D:128]
            out_ref[j, :, 0, :, 0:D] = a.T.reshape(8, 8, D)
            out_ref[j, :, 0, :, D:128] = b.T.reshape(8, 8, D)

    return pl.pallas_call(
        body,
        grid=(S // SB, NB),
        in_specs=[
            pl.BlockSpec((SB, 1, D, 128), lambda si, bt: (si, bt, 0, 0))
        ],
        out_specs=pl.BlockSpec(
            (SB, 8, 1, 8, 128), lambda si, bt: (si, 0, bt, 0, 0)
        ),
        out_shape=jax.ShapeDtypeStruct((S, 8, NB, 8, 128), jnp.float32),
        compiler_params=pltpu.CompilerParams(
            dimension_semantics=("parallel", "parallel")
        ),
    )(g4)


def kernel(input_ids, token_emb):
    B, S = input_ids.shape
    n = B * S
    NB = B // 128
    tab128 = _tab_rows(token_emb.T)
    tab2 = tab128.reshape(2 * VOCAB_PAD, D)
    idx2 = (
        (input_ids.astype(jnp.int32) * 2)
        .T.reshape(S, NB, 2, D)
        .transpose(0, 1, 3, 2)
        .reshape(1, n)
    )
    g = _sc_gather(tab2, idx2, n)
    out5 = _out_format(g.reshape(S, NB, D, 128), S, NB)
    return out5.transpose(2, 4, 0, 1, 3).reshape(B, S, D)


# speedup vs baseline: 2.0885x; 2.0885x over previous
import functools

import jax
import jax.numpy as jnp
from jax.experimental import pallas as pl
from jax.experimental.pallas import tpu as pltpu
from jax.experimental.pallas import tpu_sc as plsc

VOCAB = 1000000
D = 64
TAB_WB = 32768
VOCAB_PAD = 31 * 32768
GATHER_W = 256


def _tab_rows(tabT):
    def body(in_ref, out_ref):
        out_ref[:, 0:D] = in_ref[...].T

    return pl.pallas_call(
        body,
        grid=(VOCAB_PAD // TAB_WB,),
        in_specs=[pl.BlockSpec((D, TAB_WB), lambda i: (0, i))],
        out_specs=pl.BlockSpec((TAB_WB, 128), lambda i: (i, 0)),
        out_shape=jax.ShapeDtypeStruct((VOCAB_PAD, 128), jnp.float32),
        compiler_params=pltpu.CompilerParams(
            dimension_semantics=("parallel",)
        ),
    )(tabT)


def _sc_gather(tab128, idx_flat, n):
    mesh = plsc.VectorSubcoreMesh(core_axis_name="c", subcore_axis_name="s")

    @functools.partial(
        pl.kernel,
        out_type=jax.ShapeDtypeStruct((n, 128), jnp.float32),
        mesh=mesh,
        compiler_params=pltpu.CompilerParams(use_tc_tiling_on_sc=False),
    )
    def gather_kernel(x_hbm, i_hbm, o_hbm):
        def body(i_vmem, o_vmem):
            pltpu.sync_copy(x_hbm.at[i_vmem.at[0]], o_vmem)

        pltpu.emit_pipeline(
            body,
            grid=(n // GATHER_W,),
            in_specs=[pl.BlockSpec((1, GATHER_W), lambda i: (0, i))],
            out_specs=[pl.BlockSpec((GATHER_W, 128), lambda i: (i, 0))],
            core_axis_name=("c", "s"),
            dimension_semantics=(pltpu.PARALLEL,),
        )(i_hbm, o_hbm)

    return gather_kernel(tab128, idx_flat)


SB = 100


def _out_format(g4, S, NB):
    def body(in_ref, out_ref):
        for j in range(SB):
            tok = in_ref[j, 0, :, 0:D]
            out_ref[j, :, 0, :, :] = tok.T.reshape(8, 8, 128)

    return pl.pallas_call(
        body,
        grid=(S // SB, NB),
        in_specs=[
            pl.BlockSpec((SB, 1, 128, 128), lambda si, bt: (si, bt, 0, 0))
        ],
        out_specs=pl.BlockSpec(
            (SB, 8, 1, 8, 128), lambda si, bt: (si, 0, bt, 0, 0)
        ),
        out_shape=jax.ShapeDtypeStruct((S, 8, NB, 8, 128), jnp.float32),
        compiler_params=pltpu.CompilerParams(
            dimension_semantics=("parallel", "parallel")
        ),
    )(g4)


def kernel(input_ids, token_emb):
    B, S = input_ids.shape
    n = B * S
    NB = B // 128
    tab128 = _tab_rows(token_emb.T)
    idx = input_ids.T.astype(jnp.int32).reshape(1, n)
    g = _sc_gather(tab128, idx, n)
    out5 = _out_format(g.reshape(S, NB, 128, 128), S, NB)
    return out5.transpose(2, 4, 0, 1, 3).reshape(B, S, D)
